# Initial kernel scaffold; baseline (speedup 1.0000x reference)
import jax
import jax.numpy as jnp
from jax.experimental import pallas as pl


def kernel(x, tok_emb, pos_emb):
    raise NotImplementedError("write your pallas kernel here")


# speedup vs baseline: 3.7000x; 3.7000x over previous
import functools

import jax
import jax.numpy as jnp
from jax import lax
from jax.experimental import pallas as pl
from jax.experimental.pallas import tpu as pltpu
from jax.experimental.pallas import tpu_sc as plsc

N_EMBD = 64
SEQ = 200
BATCH = 4096
N_ROWS = BATCH * SEQ

_INFO = plsc.get_sparse_core_info()
NC, NS, L = _INFO.num_cores, _INFO.num_subcores, _INFO.num_lanes
NW = NC * NS

SEQ_PER_WORKER = BATCH // NW
CHUNK_SEQS = 4
CHUNK_ROWS = CHUNK_SEQS * SEQ
CHUNKS = SEQ_PER_WORKER // CHUNK_SEQS
ROWS_PER_WORKER = SEQ_PER_WORKER * SEQ

_mesh = plsc.VectorSubcoreMesh(core_axis_name="c", subcore_axis_name="s")


@functools.partial(
    pl.kernel,
    mesh=_mesh,
    out_type=jax.ShapeDtypeStruct((N_ROWS, N_EMBD), jnp.float32),
    scratch_types=[
        pltpu.VMEM((SEQ, N_EMBD), jnp.float32),
        pltpu.VMEM((CHUNK_ROWS,), jnp.int32),
        pltpu.VMEM((CHUNK_ROWS, N_EMBD), jnp.float32),
        pltpu.SemaphoreType.DMA,
    ],
    compiler_params=pltpu.CompilerParams(use_tc_tiling_on_sc=False),
)
def _emb_lookup(idx_hbm, tok_hbm, pos_hbm, out_hbm, pos_v, idx_v, rows_v, sem):
    wid = lax.axis_index("s") * NC + lax.axis_index("c")
    base_row = wid * ROWS_PER_WORKER
    pltpu.sync_copy(pos_hbm, pos_v)

    def chunk_body(g, carry):
        row0 = base_row + g * CHUNK_ROWS
        pltpu.sync_copy(idx_hbm.at[pl.ds(row0, CHUNK_ROWS)], idx_v)
        pltpu.async_copy(tok_hbm.at[idx_v], rows_v, sem).wait()

        def row_body(r, carry2):
            for c in range(N_EMBD // L):
                col = pl.ds(c * L, L)
                pvec = pos_v[r, col]
                for s in range(CHUNK_SEQS):
                    rr = s * SEQ + r
                    rows_v[rr, col] = rows_v[rr, col] + pvec
            return carry2

        lax.fori_loop(0, SEQ, row_body, 0)
        pltpu.sync_copy(rows_v, out_hbm.at[pl.ds(row0, CHUNK_ROWS)])
        return carry

    lax.fori_loop(0, CHUNKS, chunk_body, 0)


def kernel(x, tok_emb, pos_emb):
    idx = x.reshape(-1).astype(jnp.int32)
    out = _emb_lookup(idx, tok_emb, pos_emb)
    return out.reshape(x.shape[0], x.shape[1], N_EMBD)


# speedup vs baseline: 4.2109x; 1.1381x over previous
import functools

import jax
import jax.numpy as jnp
from jax import lax
from jax.experimental import pallas as pl
from jax.experimental.pallas import tpu as pltpu
from jax.experimental.pallas import tpu_sc as plsc

N_EMBD = 64
SEQ = 200
BATCH = 4096
N_ROWS = BATCH * SEQ

_INFO = plsc.get_sparse_core_info()
NC, NS, L = _INFO.num_cores, _INFO.num_subcores, _INFO.num_lanes
NW = NC * NS

SEQ_PER_WORKER = BATCH // NW
CHUNK_SEQS = 2
CHUNK_ROWS = CHUNK_SEQS * SEQ
CHUNKS = SEQ_PER_WORKER // CHUNK_SEQS
ROWS_PER_WORKER = SEQ_PER_WORKER * SEQ
RING = 4

_mesh = plsc.VectorSubcoreMesh(core_axis_name="c", subcore_axis_name="s")


@functools.partial(
    pl.kernel,
    mesh=_mesh,
    out_type=jax.ShapeDtypeStruct((N_ROWS, N_EMBD), jnp.float32),
    scratch_types=[
        pltpu.VMEM((SEQ, N_EMBD), jnp.float32),
        pltpu.VMEM((RING, CHUNK_ROWS), jnp.int32),
        pltpu.VMEM((RING, CHUNK_ROWS, N_EMBD), jnp.float32),
    ]
    + [pltpu.SemaphoreType.DMA] * RING
    + [pltpu.SemaphoreType.DMA] * RING,
    compiler_params=pltpu.CompilerParams(use_tc_tiling_on_sc=False),
)
def _emb_lookup(idx_hbm, tok_hbm, pos_hbm, out_hbm, pos_v, idx_v, rows_v, *sems):
    gsem = sems[:RING]
    osem = sems[RING:]
    wid = lax.axis_index("s") * NC + lax.axis_index("c")
    base_row = wid * ROWS_PER_WORKER
    pltpu.sync_copy(pos_hbm, pos_v)

    def issue_gather(h, slot):
        row0 = base_row + h * CHUNK_ROWS
        pltpu.sync_copy(idx_hbm.at[pl.ds(row0, CHUNK_ROWS)], idx_v.at[slot])
        pltpu.async_copy(tok_hbm.at[idx_v.at[slot]], rows_v.at[slot], gsem[slot])

    issue_gather(0, 0)
    issue_gather(1, 1)

    def group_body(gg, carry):
        for b in range(RING):
            g = gg * RING + b
            hb = (b + 2) % RING

            @pl.when(g + 2 < CHUNKS)
            def _():
                @pl.when(g + 2 >= RING)
                def _():
                    pltpu.make_async_copy(
                        rows_v.at[hb],
                        out_hbm.at[pl.ds(base_row, CHUNK_ROWS)],
                        osem[hb],
                    ).wait()
                issue_gather(g + 2, hb)

            pltpu.make_async_copy(
                tok_hbm.at[idx_v.at[b]], rows_v.at[b], gsem[b]
            ).wait()

            def row_body(r, carry2):
                for c in range(N_EMBD // L):
                    col = pl.ds(c * L, L)
                    pvec = pos_v[r, col]
                    for s in range(CHUNK_SEQS):
                        rr = s * SEQ + r
                        rows_v[b, rr, col] = rows_v[b, rr, col] + pvec
                return carry2

            lax.fori_loop(0, SEQ, row_body, 0)

            pltpu.async_copy(
                rows_v.at[b],
                out_hbm.at[pl.ds(base_row + g * CHUNK_ROWS, CHUNK_ROWS)],
                osem[b],
            )
        return carry

    lax.fori_loop(0, CHUNKS // RING, group_body, 0)

    for b in range(RING):
        pltpu.make_async_copy(
            rows_v.at[b],
            out_hbm.at[pl.ds(base_row, CHUNK_ROWS)],
            osem[b],
        ).wait()


def kernel(x, tok_emb, pos_emb):
    idx = x.reshape(-1).astype(jnp.int32)
    out = _emb_lookup(idx, tok_emb, pos_emb)
    return out.reshape(x.shape[0], x.shape[1], N_EMBD)
